# baseline (speedup 1.0000x reference)
import functools

import jax
import jax.numpy as jnp
from jax.experimental import pallas as pl
from jax.experimental.pallas import tpu as pltpu

ALPHA = 0.25
GAMMA = 2.0
W_CE = 2.0
W_BBOX = 5.0
W_GIOU = 2.0


def _xyxy_cols(bx):
    cx = bx[:, 0:1]
    cy = bx[:, 1:2]
    w = bx[:, 2:3]
    h = bx[:, 3:4]
    return cx - 0.5 * w, cy - 0.5 * h, cx + 0.5 * w, cy + 0.5 * h


def _loss_kernel(x_ref, boxes_ref, tb_ref, sidx_col_ref, lab_col_ref,
                 sidx_row_ref, lab_row_ref, out_ref, *, q, c, t_per_b, nb):
    x = x_ref[...]
    boxes = boxes_ref[...]
    tb = tb_ref[...]
    sidx_col = sidx_col_ref[...]
    lab_col = lab_col_ref[...]
    sidx_row = sidx_row_ref[...]
    lab_row = lab_row_ref[...]

    bq = x.shape[0]
    n = tb.shape[0]

    iota_n0 = jax.lax.broadcasted_iota(jnp.int32, (n, 1), 0)
    flat_col = sidx_col + (iota_n0 // t_per_b) * q
    iota_n1 = jax.lax.broadcasted_iota(jnp.int32, (1, n), 1)
    flat_row = sidx_row + (iota_n1 // t_per_b) * q

    iota_bq1 = jax.lax.broadcasted_iota(jnp.int32, (n, bq), 1)
    oh = (flat_col == iota_bq1).astype(jnp.float32)

    prob = jax.nn.sigmoid(x)
    ce0 = jnp.maximum(x, 0.0) + jnp.log1p(jnp.exp(-jnp.abs(x)))
    loss0_sum = jnp.sum((1.0 - ALPHA) * prob * prob * ce0)

    rows = jax.lax.dot_general(oh, x, (((1,), (0,)), ((), ())),
                               preferred_element_type=jnp.float32,
                               precision=jax.lax.Precision.HIGHEST)
    iota_c1 = jax.lax.broadcasted_iota(jnp.int32, (n, c), 1)
    oh_tc = (lab_col == iota_c1).astype(jnp.float32)
    v = jnp.sum(rows * oh_tc, axis=1, keepdims=True)

    key_col = flat_col * c + lab_col
    key_row = flat_row * c + lab_row
    earlier = iota_n1 < iota_n0
    dup = jnp.max(jnp.where((key_col == key_row) & earlier, 1.0, 0.0),
                  axis=1, keepdims=True)
    keep = 1.0 - dup

    pv = jax.nn.sigmoid(v)
    sp_neg = jnp.log1p(jnp.exp(-jnp.abs(v)))
    ce1 = jnp.maximum(v, 0.0) - v + sp_neg
    ce0v = jnp.maximum(v, 0.0) + sp_neg
    one_m = 1.0 - pv
    delta = ALPHA * one_m * one_m * ce1 - (1.0 - ALPHA) * pv * pv * ce0v
    corr_sum = jnp.sum(keep * delta)

    ce_l = (loss0_sum + corr_sum) / nb

    pb = jax.lax.dot_general(oh, boxes, (((1,), (0,)), ((), ())),
                             preferred_element_type=jnp.float32,
                             precision=jax.lax.Precision.HIGHEST)
    bb_l = jnp.sum(jnp.abs(pb - tb)) / nb

    px1, py1, px2, py2 = _xyxy_cols(pb)
    tx1, ty1, tx2, ty2 = _xyxy_cols(tb)
    area_p = (px2 - px1) * (py2 - py1)
    area_t = (tx2 - tx1) * (ty2 - ty1)
    iw = jnp.clip(jnp.minimum(px2, tx2) - jnp.maximum(px1, tx1), 0.0, None)
    ih = jnp.clip(jnp.minimum(py2, ty2) - jnp.maximum(py1, ty1), 0.0, None)
    inter = iw * ih
    union = area_p + area_t - inter
    iou = inter / union
    ew = jnp.clip(jnp.maximum(px2, tx2) - jnp.minimum(px1, tx1), 0.0, None)
    eh = jnp.clip(jnp.maximum(py2, ty2) - jnp.minimum(py1, ty1), 0.0, None)
    earea = ew * eh
    g = iou - (earea - union) / earea
    gi_l = jnp.sum(1.0 - g) / nb

    out_ref[0] = ce_l
    out_ref[1] = bb_l
    out_ref[2] = gi_l
    out_ref[3] = W_CE * ce_l + W_BBOX * bb_l + W_GIOU * gi_l


def kernel(pred_logits, pred_boxes, tgt_boxes, tgt_labels, src_idx):
    B, Q, C = pred_logits.shape
    T = tgt_labels.shape[1]
    N = B * T
    nb = float(max(1, N))

    x2 = pred_logits.reshape(B * Q, C)
    boxes2 = pred_boxes.reshape(B * Q, 4)
    tb2 = tgt_boxes.reshape(N, 4)
    sidx_col = src_idx.reshape(N, 1).astype(jnp.int32)
    lab_col = tgt_labels.reshape(N, 1).astype(jnp.int32)
    sidx_row = src_idx.reshape(1, N).astype(jnp.int32)
    lab_row = tgt_labels.reshape(1, N).astype(jnp.int32)

    out = pl.pallas_call(
        functools.partial(_loss_kernel, q=Q, c=C, t_per_b=T, nb=nb),
        out_specs=pl.BlockSpec(memory_space=pltpu.SMEM),
        out_shape=jax.ShapeDtypeStruct((4,), jnp.float32),
    )(x2, boxes2, tb2, sidx_col, lab_col, sidx_row, lab_row)

    return (out[0], out[1], out[2], out[3])


# speedup vs baseline: 1.3490x; 1.3490x over previous
import functools

import jax
import jax.numpy as jnp
from jax.experimental import pallas as pl
from jax.experimental.pallas import tpu as pltpu

ALPHA = 0.25
GAMMA = 2.0
W_CE = 2.0
W_BBOX = 5.0
W_GIOU = 2.0


def _xyxy_cols(bx):
    cx = bx[:, 0:1]
    cy = bx[:, 1:2]
    w = bx[:, 2:3]
    h = bx[:, 3:4]
    return cx - 0.5 * w, cy - 0.5 * h, cx + 0.5 * w, cy + 0.5 * h


def _loss_kernel(x_ref, boxes_ref, tb_ref, sidx_ref, lab_ref,
                 sidx_r_ref, lab_r_ref, out_ref, *, nb):
    x = x_ref[...]
    nbatch, q, c = x.shape
    t = tb_ref.shape[1]

    prob = jax.nn.sigmoid(x)
    ce0 = jnp.maximum(x, 0.0) + jnp.log1p(jnp.exp(-jnp.abs(x)))
    loss0_sum = jnp.sum((1.0 - ALPHA) * prob * prob * ce0)

    iota_q1 = jax.lax.broadcasted_iota(jnp.int32, (t, q), 1)
    iota_c1 = jax.lax.broadcasted_iota(jnp.int32, (t, c), 1)
    iota_t0 = jax.lax.broadcasted_iota(jnp.int32, (t, t), 0)
    iota_t1 = jax.lax.broadcasted_iota(jnp.int32, (t, t), 1)
    earlier = iota_t1 < iota_t0

    corr_sum = 0.0
    l1_sum = 0.0
    giou_sum = 0.0
    for b in range(nbatch):
        sidx_b = sidx_ref[b]
        lab_b = lab_ref[b]
        oh_tq = (sidx_b == iota_q1).astype(jnp.float32)
        oh_tc = (lab_b == iota_c1).astype(jnp.float32)

        rows = jax.lax.dot_general(oh_tq, x_ref[b], (((1,), (0,)), ((), ())),
                                   preferred_element_type=jnp.float32,
                                   precision=jax.lax.Precision.HIGHEST)
        v = jnp.sum(rows * oh_tc, axis=1, keepdims=True)

        key_col = sidx_b * c + lab_b
        key_row = sidx_r_ref[b] * c + lab_r_ref[b]
        dup = jnp.max(jnp.where((key_col == key_row) & earlier, 1.0, 0.0),
                      axis=1, keepdims=True)
        keep = 1.0 - dup

        pv = jax.nn.sigmoid(v)
        sp = jnp.log1p(jnp.exp(-jnp.abs(v)))
        ce1 = jnp.maximum(v, 0.0) - v + sp
        ce0v = jnp.maximum(v, 0.0) + sp
        one_m = 1.0 - pv
        delta = ALPHA * one_m * one_m * ce1 - (1.0 - ALPHA) * pv * pv * ce0v
        corr_sum = corr_sum + jnp.sum(keep * delta)

        pb = jax.lax.dot_general(oh_tq, boxes_ref[b], (((1,), (0,)), ((), ())),
                                 preferred_element_type=jnp.float32,
                                 precision=jax.lax.Precision.HIGHEST)
        tb = tb_ref[b]
        l1_sum = l1_sum + jnp.sum(jnp.abs(pb - tb))

        px1, py1, px2, py2 = _xyxy_cols(pb)
        tx1, ty1, tx2, ty2 = _xyxy_cols(tb)
        area_p = (px2 - px1) * (py2 - py1)
        area_t = (tx2 - tx1) * (ty2 - ty1)
        iw = jnp.clip(jnp.minimum(px2, tx2) - jnp.maximum(px1, tx1), 0.0, None)
        ih = jnp.clip(jnp.minimum(py2, ty2) - jnp.maximum(py1, ty1), 0.0, None)
        inter = iw * ih
        union = area_p + area_t - inter
        iou = inter / union
        ew = jnp.clip(jnp.maximum(px2, tx2) - jnp.minimum(px1, tx1), 0.0, None)
        eh = jnp.clip(jnp.maximum(py2, ty2) - jnp.minimum(py1, ty1), 0.0, None)
        earea = ew * eh
        g = iou - (earea - union) / earea
        giou_sum = giou_sum + jnp.sum(1.0 - g)

    ce_l = (loss0_sum + corr_sum) / nb
    bb_l = l1_sum / nb
    gi_l = giou_sum / nb
    out_ref[0] = ce_l
    out_ref[1] = bb_l
    out_ref[2] = gi_l
    out_ref[3] = W_CE * ce_l + W_BBOX * bb_l + W_GIOU * gi_l


def kernel(pred_logits, pred_boxes, tgt_boxes, tgt_labels, src_idx):
    B, Q, C = pred_logits.shape
    T = tgt_labels.shape[1]
    nb = float(max(1, B * T))

    sidx3 = src_idx.reshape(B, T, 1).astype(jnp.int32)
    lab3 = tgt_labels.reshape(B, T, 1).astype(jnp.int32)
    sidx_r3 = src_idx.reshape(B, 1, T).astype(jnp.int32)
    lab_r3 = tgt_labels.reshape(B, 1, T).astype(jnp.int32)

    out = pl.pallas_call(
        functools.partial(_loss_kernel, nb=nb),
        out_specs=pl.BlockSpec(memory_space=pltpu.SMEM),
        out_shape=jax.ShapeDtypeStruct((4,), jnp.float32),
    )(pred_logits, pred_boxes, tgt_boxes, sidx3, lab3, sidx_r3, lab_r3)

    return (out[0], out[1], out[2], out[3])


# speedup vs baseline: 1.4377x; 1.0658x over previous
import functools

import jax
import jax.numpy as jnp
from jax.experimental import pallas as pl
from jax.experimental.pallas import tpu as pltpu

ALPHA = 0.25
GAMMA = 2.0
W_CE = 2.0
W_BBOX = 5.0
W_GIOU = 2.0


def _xyxy_cols(bx):
    cx = bx[:, 0:1]
    cy = bx[:, 1:2]
    w = bx[:, 2:3]
    h = bx[:, 3:4]
    return cx - 0.5 * w, cy - 0.5 * h, cx + 0.5 * w, cy + 0.5 * h


def _loss_kernel(x_ref, boxes_ref, tb_ref, sidx_ref, lab_ref,
                 sidx_r_ref, lab_r_ref, out_ref, *, nb):
    x = x_ref[...]
    nbatch, q, c = x.shape
    t = tb_ref.shape[1]

    x_bf = x.astype(jnp.bfloat16)
    boxes_bf = boxes_ref[...].astype(jnp.bfloat16)

    iota_q1 = jax.lax.broadcasted_iota(jnp.int32, (t, q), 1)
    iota_c1 = jax.lax.broadcasted_iota(jnp.int32, (t, c), 1)
    iota_t0 = jax.lax.broadcasted_iota(jnp.int32, (t, t), 0)
    iota_t1 = jax.lax.broadcasted_iota(jnp.int32, (t, t), 1)
    earlier = iota_t1 < iota_t0

    v_cols = []
    keep_cols = []
    l1_sum = 0.0
    giou_sum = 0.0
    for b in range(nbatch):
        sidx_b = sidx_ref[b]
        lab_b = lab_ref[b]
        oh_tq = (sidx_b == iota_q1).astype(jnp.bfloat16)

        rows = jax.lax.dot_general(oh_tq, x_bf[b], (((1,), (0,)), ((), ())),
                                   preferred_element_type=jnp.float32)
        v_cols.append(jnp.sum(jnp.where(lab_b == iota_c1, rows, 0.0),
                              axis=1, keepdims=True))

        key_col = sidx_b * c + lab_b
        key_row = sidx_r_ref[b] * c + lab_r_ref[b]
        dup = jnp.max(jnp.where((key_col == key_row) & earlier, 1.0, 0.0),
                      axis=1, keepdims=True)
        keep_cols.append(1.0 - dup)

        pb = jax.lax.dot_general(oh_tq, boxes_bf[b], (((1,), (0,)), ((), ())),
                                 preferred_element_type=jnp.float32)
        tb = tb_ref[b]
        l1_sum = l1_sum + jnp.sum(jnp.abs(pb - tb))

        px1, py1, px2, py2 = _xyxy_cols(pb)
        tx1, ty1, tx2, ty2 = _xyxy_cols(tb)
        area_p = (px2 - px1) * (py2 - py1)
        area_t = (tx2 - tx1) * (ty2 - ty1)
        iw = jnp.clip(jnp.minimum(px2, tx2) - jnp.maximum(px1, tx1), 0.0, None)
        ih = jnp.clip(jnp.minimum(py2, ty2) - jnp.maximum(py1, ty1), 0.0, None)
        inter = iw * ih
        union = area_p + area_t - inter
        iou = inter / union
        ew = jnp.clip(jnp.maximum(px2, tx2) - jnp.minimum(px1, tx1), 0.0, None)
        eh = jnp.clip(jnp.maximum(py2, ty2) - jnp.minimum(py1, ty1), 0.0, None)
        earea = ew * eh
        g = iou - (earea - union) / earea
        giou_sum = giou_sum + jnp.sum(1.0 - g)

    e = jnp.exp(-jnp.abs(x))
    lse = jnp.log1p(e)
    r = 1.0 / (1.0 + e)
    p = jnp.where(x >= 0.0, r, 1.0 - r)
    ce0 = jnp.maximum(x, 0.0) + lse
    loss0_sum = jnp.sum((1.0 - ALPHA) * (p * p) * ce0)

    v = jnp.concatenate(v_cols, axis=1)
    keep = jnp.concatenate(keep_cols, axis=1)
    ev = jnp.exp(-jnp.abs(v))
    lsev = jnp.log1p(ev)
    rv = 1.0 / (1.0 + ev)
    pv = jnp.where(v >= 0.0, rv, 1.0 - rv)
    ce0v = jnp.maximum(v, 0.0) + lsev
    l0v = (1.0 - ALPHA) * (pv * pv) * ce0v
    omv = 1.0 - pv
    delta = ALPHA * (omv * omv) * (ce0v - v) - l0v
    corr_sum = jnp.sum(keep * delta)

    ce_l = (loss0_sum + corr_sum) / nb
    bb_l = l1_sum / nb
    gi_l = giou_sum / nb
    out_ref[0] = ce_l
    out_ref[1] = bb_l
    out_ref[2] = gi_l
    out_ref[3] = W_CE * ce_l + W_BBOX * bb_l + W_GIOU * gi_l


def kernel(pred_logits, pred_boxes, tgt_boxes, tgt_labels, src_idx):
    B, Q, C = pred_logits.shape
    T = tgt_labels.shape[1]
    nb = float(max(1, B * T))

    sidx3 = src_idx.reshape(B, T, 1).astype(jnp.int32)
    lab3 = tgt_labels.reshape(B, T, 1).astype(jnp.int32)
    sidx_r3 = src_idx.reshape(B, 1, T).astype(jnp.int32)
    lab_r3 = tgt_labels.reshape(B, 1, T).astype(jnp.int32)

    out = pl.pallas_call(
        functools.partial(_loss_kernel, nb=nb),
        out_specs=pl.BlockSpec(memory_space=pltpu.SMEM),
        out_shape=jax.ShapeDtypeStruct((4,), jnp.float32),
    )(pred_logits, pred_boxes, tgt_boxes, sidx3, lab3, sidx_r3, lab_r3)

    return (out[0], out[1], out[2], out[3])
